# baseline (speedup 1.0000x reference)
import functools

import jax
import jax.numpy as jnp
from jax import lax
from jax.experimental import pallas as pl
from jax.experimental.pallas import tpu as pltpu
from jax.experimental.pallas import tpu_sc as plsc

B = 4096
D = 64
V = 100000
_PACK_C = 8192
_Q = _PACK_C // 4

_NBLK = -(-V // _PACK_C)
_VPAD = _NBLK * _Q


def _rne16(x):
    xi = lax.bitcast_convert_type(x, jnp.int32)
    r = (xi + 0x7FFF + ((xi >> 16) & 1)) >> 16
    return r & 0xFFFF


def _pack_body(t_ref, eye_ref, o_ref):
    x = t_ref[...]
    col = pl.program_id(0) * _PACK_C + lax.broadcasted_iota(
        jnp.int32, (D, _PACK_C), 1)
    x = jnp.where(col < V, x, 0.0)
    xs = jnp.concatenate(
        [x[:, j * _Q:(j + 1) * _Q] for j in range(4)], axis=0)
    xt = lax.dot_general(xs, eye_ref[...], (((0,), (0,)), ((), ())))
    words = []
    for j in range(4):
        lo = _rne16(xt[:, j * D:j * D + 32])
        hi = _rne16(xt[:, j * D + 32:(j + 1) * D])
        words.append(lo | (hi << 16))
    w = jnp.concatenate(words, axis=1)
    o_ref[...] = lax.bitcast_convert_type(w, jnp.float32)


def _pack(table_t, eye):
    return pl.pallas_call(
        _pack_body,
        grid=(_NBLK,),
        in_specs=[
            pl.BlockSpec((D, _PACK_C), lambda k: (0, k)),
            pl.BlockSpec((4 * D, 4 * D), lambda k: (0, 0)),
        ],
        out_specs=pl.BlockSpec((_Q, 2 * D), lambda k: (k, 0)),
        out_shape=jax.ShapeDtypeStruct((_VPAD, 2 * D), jnp.float32),
    )(table_t, eye)


def _row_quarter(idx):
    row = (idx // _PACK_C) * _Q + (idx % _Q)
    return row, ((idx % _PACK_C) // _Q).astype(jnp.float32)


@functools.lru_cache(maxsize=None)
def _build_sc_gather():
    info = plsc.get_sparse_core_info()
    nc, ns = info.num_cores, info.num_subcores
    nw = nc * ns
    b_per_w = B // nw

    mesh = plsc.VectorSubcoreMesh(core_axis_name="c", subcore_axis_name="s")

    @functools.partial(
        pl.kernel,
        mesh=mesh,
        out_type=jax.ShapeDtypeStruct((B, 2 * D), jnp.float32),
        scratch_types=[
            pltpu.VMEM((b_per_w,), jnp.int32),
            pltpu.VMEM((b_per_w, 2 * D), jnp.float32),
            pltpu.SemaphoreType.DMA,
        ],
    )
    def gather_kernel(idx_hbm, tab_hbm, rows_out, idx_v, rows_v, sem):
        wid = lax.axis_index("s") * nc + lax.axis_index("c")
        base = wid * b_per_w
        pltpu.sync_copy(idx_hbm.at[pl.ds(base, b_per_w)], idx_v)
        pltpu.async_copy(tab_hbm.at[idx_v], rows_v, sem).wait()
        pltpu.sync_copy(rows_v, rows_out.at[pl.ds(base, b_per_w)])

    return gather_kernel


def _unpack_into(w, q, s_ref):
    word = jnp.where(q == 0, w[0:32],
                     jnp.where(q == 1, w[32:64],
                               jnp.where(q == 2, w[64:96], w[96:128])))
    wi = lax.bitcast_convert_type(word, jnp.int32)
    s_ref[0:32, :] = lax.bitcast_convert_type(wi << 16, jnp.float32)
    s_ref[32:64, :] = lax.bitcast_convert_type((wi >> 16) << 16, jnp.float32)


def _dense_body(uw_ref, vw_ref, qu_ref, qv_ref, o_ref, ut_s, vt_s):
    k = pl.program_id(0)

    @pl.when(k == 0)
    def _():
        _unpack_into(uw_ref[...], qu_ref[...], ut_s)
        _unpack_into(vw_ref[...], qv_ref[...], vt_s)

    bi = o_ref.shape[0]
    ut = ut_s[pl.ds(k * bi, bi), :]
    vt = vt_s[...]
    x = ut[:, None, :] * vt[None, :, :]
    o_ref[...] = 0.5 * jnp.tanh(0.5 * x) + 0.5


def kernel(inputs, user_table, item_table):
    u_idx = inputs[:, 0]
    i_idx = inputs[:, 1]
    u_row, qu = _row_quarter(u_idx)
    i_row, qv = _row_quarter(i_idx)
    qu = qu.reshape(1, B)
    qv = qv.reshape(1, B)

    eye = jnp.eye(4 * D, dtype=jnp.float32)
    gather = _build_sc_gather()
    utab2 = _pack(user_table.T, eye)
    u_wide = gather(u_row, utab2)
    itab2 = _pack(item_table.T, eye)
    uw_t = u_wide.T
    i_wide = gather(i_row, itab2)
    vw_t = i_wide.T

    bi = 8
    out_t = pl.pallas_call(
        _dense_body,
        grid=(D // bi,),
        in_specs=[
            pl.BlockSpec((2 * D, B), lambda k: (0, 0)),
            pl.BlockSpec((2 * D, B), lambda k: (0, 0)),
            pl.BlockSpec((1, B), lambda k: (0, 0)),
            pl.BlockSpec((1, B), lambda k: (0, 0)),
        ],
        out_specs=pl.BlockSpec((bi, D, B), lambda k: (k, 0, 0)),
        out_shape=jax.ShapeDtypeStruct((D, D, B), jnp.float32),
        scratch_shapes=[
            pltpu.VMEM((D, B), jnp.float32),
            pltpu.VMEM((D, B), jnp.float32),
        ],
    )(uw_t, vw_t, qu, qv)
    return jnp.transpose(out_t, (2, 0, 1))


# speedup vs baseline: 1.1380x; 1.1380x over previous
import functools

import jax
import jax.numpy as jnp
from jax import lax
from jax.experimental import pallas as pl
from jax.experimental.pallas import tpu as pltpu
from jax.experimental.pallas import tpu_sc as plsc

B = 4096
D = 64
V = 100000
_PACK_C = 8192


_NBLK = -(-V // _PACK_C)
_VPAD = _NBLK * (_PACK_C // 2)


def _pack_body(t_ref, eye_ref, o_ref):
    x = t_ref[...]
    h = _PACK_C // 2
    col = pl.program_id(0) * _PACK_C + lax.broadcasted_iota(
        jnp.int32, (D, _PACK_C), 1)
    x = jnp.where(col < V, x, 0.0)
    xs = jnp.concatenate([x[:, :h], x[:, h:]], axis=0)
    o_ref[...] = lax.dot_general(xs, eye_ref[...], (((0,), (0,)), ((), ())))


def _pack(table_t, eye):
    return pl.pallas_call(
        _pack_body,
        grid=(_NBLK,),
        in_specs=[
            pl.BlockSpec((D, _PACK_C), lambda k: (0, k)),
            pl.BlockSpec((2 * D, 2 * D), lambda k: (0, 0)),
        ],
        out_specs=pl.BlockSpec((_PACK_C // 2, 2 * D), lambda k: (k, 0)),
        out_shape=jax.ShapeDtypeStruct((_VPAD, 2 * D), jnp.float32),
    )(table_t, eye)


def _row_half(idx):
    half = _PACK_C // 2
    row = (idx // _PACK_C) * half + (idx % half)
    return row, ((idx % _PACK_C) // half).astype(jnp.float32)


@functools.lru_cache(maxsize=None)
def _build_sc_gather():
    info = plsc.get_sparse_core_info()
    nc, ns = info.num_cores, info.num_subcores
    nw = nc * ns
    b_per_w = B // nw

    mesh = plsc.VectorSubcoreMesh(core_axis_name="c", subcore_axis_name="s")

    @functools.partial(
        pl.kernel,
        mesh=mesh,
        out_type=jax.ShapeDtypeStruct((B, 2 * D), jnp.float32),
        scratch_types=[
            pltpu.VMEM((b_per_w,), jnp.int32),
            pltpu.VMEM((b_per_w, 2 * D), jnp.float32),
            pltpu.SemaphoreType.DMA,
        ],
    )
    def gather_kernel(idx_hbm, tab_hbm, rows_out, idx_v, rows_v, sem):
        wid = lax.axis_index("s") * nc + lax.axis_index("c")
        base = wid * b_per_w
        pltpu.sync_copy(idx_hbm.at[pl.ds(base, b_per_w)], idx_v)
        pltpu.async_copy(tab_hbm.at[idx_v], rows_v, sem).wait()
        pltpu.sync_copy(rows_v, rows_out.at[pl.ds(base, b_per_w)])

    return gather_kernel


def _dense_body(ua_ref, ub_ref, va_ref, vb_ref, pu_ref, pv_ref, o_ref):
    pu = pu_ref[...]
    pv = pv_ref[...]
    ut = ua_ref[...] * (1.0 - pu) + ub_ref[...] * pu
    vt = va_ref[...] * (1.0 - pv) + vb_ref[...] * pv
    x = ut[:, None, :] * vt[None, :, :]
    o_ref[...] = 0.5 * jnp.tanh(0.5 * x) + 0.5


def kernel(inputs, user_table, item_table):
    u_idx = inputs[:, 0]
    i_idx = inputs[:, 1]
    u_row, pu = _row_half(u_idx)
    i_row, pv = _row_half(i_idx)
    pu = pu.reshape(1, B)
    pv = pv.reshape(1, B)

    eye = jnp.eye(2 * D, dtype=jnp.float32)
    gather = _build_sc_gather()
    utab2 = _pack(user_table.T, eye)
    u_wide = gather(u_row, utab2)
    itab2 = _pack(item_table.T, eye)
    uw_t = u_wide.T
    i_wide = gather(i_row, itab2)
    vw_t = i_wide.T

    bi = 8
    out_t = pl.pallas_call(
        _dense_body,
        grid=(D // bi,),
        in_specs=[
            pl.BlockSpec((bi, B), lambda k: (k, 0)),
            pl.BlockSpec((bi, B), lambda k: (k + D // bi, 0)),
            pl.BlockSpec((D, B), lambda k: (0, 0)),
            pl.BlockSpec((D, B), lambda k: (1, 0)),
            pl.BlockSpec((1, B), lambda k: (0, 0)),
            pl.BlockSpec((1, B), lambda k: (0, 0)),
        ],
        out_specs=pl.BlockSpec((bi, D, B), lambda k: (k, 0, 0)),
        out_shape=jax.ShapeDtypeStruct((D, D, B), jnp.float32),
    )(uw_t, uw_t, vw_t, vw_t, pu, pv)
    return jnp.transpose(out_t, (2, 0, 1))
